# baseline (speedup 1.0000x reference)
import functools
import jax
import jax.numpy as jnp
from jax import lax
from jax.experimental import pallas as pl
from jax.experimental.pallas import tpu as pltpu
from jax.experimental.pallas import tpu_sc as plsc

_E = 64
_NEG = -3.0e38


def _matmul_kernel(x_ref, w_ref, l_ref):
    l_ref[...] = jax.lax.dot_general(
        w_ref[...], x_ref[...], (((1,), (1,)), ((), ())),
        preferred_element_type=jnp.float32)


def _tc_logits(x, W):
    N, D = x.shape
    T = 2048
    return pl.pallas_call(
        _matmul_kernel,
        grid=(N // T,),
        in_specs=[
            pl.BlockSpec((T, D), lambda i: (i, 0)),
            pl.BlockSpec((_E, D), lambda i: (0, 0)),
        ],
        out_specs=pl.BlockSpec((_E, T), lambda i: (0, i)),
        out_shape=jax.ShapeDtypeStruct((_E, N), jnp.float32),
    )(x, W)


def _make_sc_epilogue(N):
    info = plsc.get_sparse_core_info()
    NC, NS, L = info.num_cores, info.num_subcores, info.num_lanes
    NW = NC * NS
    TOKW = N // NW
    mesh = plsc.VectorSubcoreMesh(core_axis_name="c", subcore_axis_name="s")

    @functools.partial(
        pl.kernel, mesh=mesh,
        out_type=[
            jax.ShapeDtypeStruct((2, N), jnp.float32),
            jax.ShapeDtypeStruct((2, N), jnp.int32),
        ],
        scratch_types=[
            pltpu.VMEM((_E, TOKW), jnp.float32),
            pltpu.VMEM((2, TOKW), jnp.float32),
            pltpu.VMEM((2, TOKW), jnp.int32),
        ],
    )
    def sc_epilogue(l_hbm, s_hbm, i_hbm, l_v, s_v, i_v):
        wid = lax.axis_index("s") * NC + lax.axis_index("c")
        base = wid * TOKW
        pltpu.sync_copy(l_hbm.at[:, pl.ds(base, TOKW)], l_v)

        def body(g, carry):
            off = g * L
            neg = jnp.full((L,), _NEG, jnp.float32)
            m1 = neg
            m2 = neg
            i1 = jnp.zeros((L,), jnp.int32)
            i2 = jnp.zeros((L,), jnp.int32)
            for e in range(_E):
                v = l_v[e, pl.ds(off, L)]
                gt1 = v > m1
                gt2 = v > m2
                ec = jnp.full((L,), e, jnp.int32)
                i2 = jnp.where(gt1, i1, jnp.where(gt2, ec, i2))
                m2 = jnp.where(gt1, m1, jnp.where(gt2, v, m2))
                i1 = jnp.where(gt1, ec, i1)
                m1 = jnp.where(gt1, v, m1)
            z = jnp.zeros((L,), jnp.float32)
            for e in range(_E):
                z = z + jnp.exp(l_v[e, pl.ds(off, L)] - m1)
            s_v[0, pl.ds(off, L)] = 1.0 / z
            s_v[1, pl.ds(off, L)] = jnp.exp(m2 - m1) / z
            i_v[0, pl.ds(off, L)] = i1
            i_v[1, pl.ds(off, L)] = i2
            return carry

        lax.fori_loop(0, TOKW // L, body, 0)
        pltpu.sync_copy(s_v, s_hbm.at[:, pl.ds(base, TOKW)])
        pltpu.sync_copy(i_v, i_hbm.at[:, pl.ds(base, TOKW)])

    return sc_epilogue


def kernel(hidden_states, W):
    B, S, D = hidden_states.shape
    N = B * S
    x = hidden_states.reshape(N, D)
    logits_t = _tc_logits(x, W)
    s_t, i_t = _make_sc_epilogue(N)(logits_t)
    scores = s_t.T.reshape(B, S, 2)
    indices = i_t.T.reshape(B, S, 2)
    return scores, indices


# speedup vs baseline: 1.1605x; 1.1605x over previous
import jax
import jax.numpy as jnp
from jax.experimental import pallas as pl
from jax.experimental.pallas import tpu as pltpu


def _router_kernel(x_ref, w_ref, s_ref, i_ref):
    x = x_ref[...]
    w = w_ref[...]
    logits = jax.lax.dot_general(
        x, w, (((1,), (1,)), ((), ())),
        preferred_element_type=jnp.float32)
    e = logits.shape[-1]
    m = jnp.max(logits, axis=-1, keepdims=True)
    z = jnp.sum(jnp.exp(logits - m), axis=-1, keepdims=True)
    iota = jax.lax.broadcasted_iota(jnp.int32, logits.shape, 1)
    big = jnp.int32(e)
    idx1 = jnp.min(jnp.where(logits == m, iota, big), axis=-1, keepdims=True)
    masked = jnp.where(iota == idx1, -jnp.inf, logits)
    m2 = jnp.max(masked, axis=-1, keepdims=True)
    idx2 = jnp.min(jnp.where(masked == m2, iota, big), axis=-1, keepdims=True)
    s1 = 1.0 / z
    s2 = jnp.exp(m2 - m) / z
    s_ref[...] = jnp.concatenate([s1, s2], axis=-1)
    i_ref[...] = jnp.concatenate([idx1, idx2], axis=-1)


def kernel(hidden_states, W):
    B, S, D = hidden_states.shape
    E = W.shape[0]
    N = B * S
    x = hidden_states.reshape(N, D)
    T = 2048
    scores, indices = pl.pallas_call(
        _router_kernel,
        grid=(N // T,),
        compiler_params=pltpu.CompilerParams(
            dimension_semantics=("parallel",)),
        in_specs=[
            pl.BlockSpec((T, D), lambda i: (i, 0)),
            pl.BlockSpec((E, D), lambda i: (0, 0)),
        ],
        out_specs=[
            pl.BlockSpec((T, 2), lambda i: (i, 0)),
            pl.BlockSpec((T, 2), lambda i: (i, 0)),
        ],
        out_shape=[
            jax.ShapeDtypeStruct((N, 2), jnp.float32),
            jax.ShapeDtypeStruct((N, 2), jnp.int32),
        ],
    )(x, W)
    return scores.reshape(B, S, 2), indices.reshape(B, S, 2)
